# baseline (speedup 1.0000x reference)
import jax
import jax.numpy as jnp
from jax import lax
from jax.experimental import pallas as pl
from jax.experimental.pallas import tpu as pltpu
from jax.experimental.pallas import tpu_sc as plsc

T, B, D, C = 2048, 8, 512, 1000
BLANK = 0
TBLK = 128
CAP = 64
FCH = 64
NK = D // 16


def _argmax_body(logit_ref, pad_ref, pred_ref):
    x = logit_ref[...]
    m = jnp.max(x, axis=-1, keepdims=True)
    iota = lax.broadcasted_iota(jnp.int32, x.shape, 2)
    idx = jnp.min(jnp.where(x >= m, iota, C), axis=-1)
    pad = pad_ref[...]
    idx_t = jnp.transpose(idx, (1, 0))
    pred_ref[...] = jnp.where(pad != 0, BLANK, idx_t).astype(jnp.int32)


def _padmask_body(len_ref, mask_ref):
    l = len_ref[...][:, 0:1]
    iota = lax.broadcasted_iota(jnp.int32, (B, T), 1)
    mask_ref[...] = (iota >= l).astype(jnp.int32)


def _lane0(vec16):
    return vec16[0]


def _sc_shrink_body(rep_hbm, pred_hbm, out_hbm, gloss_hbm, len_hbm,
                    repv, predv, staging, gbuf, cnt, lenv, state):
    c = lax.axis_index("c")
    s = lax.axis_index("s")
    w = s * 2 + c
    lane_iota = lax.iota(jnp.int32, 16)
    lane0_mask = lane_iota == 0

    @pl.when(w < B)
    def _():
        b = w

        pltpu.sync_copy(pred_hbm.at[b], predv.at[pl.ds(0, T)])

        @pl.loop(0, CAP)
        def _(r):
            for k in range(NK):
                staging[r, pl.ds(k * 16, 16)] = jnp.zeros((16,), jnp.float32)
            cnt[r] = 0
        for k in range(CAP // 16):
            gbuf[pl.ds(k * 16, 16)] = jnp.zeros((16,), jnp.int32)
        state[0] = -1
        state[1] = -1
        state[2] = 0

        def flush(nused):
            def scale(r, carry):
                cf = jnp.maximum(cnt[r].astype(jnp.float32), 1.0)
                rcpv = 1.0 / (jnp.zeros((16,), jnp.float32) + cf)
                for k in range(NK):
                    sl = pl.ds(k * 16, 16)
                    staging[r, sl] = staging[r, sl] * rcpv
                return carry
            lax.fori_loop(0, nused, scale, 0)
            base = pl.multiple_of(state[2], CAP)
            pltpu.sync_copy(staging, out_hbm.at[pl.ds(base, CAP), b])
            pltpu.sync_copy(gbuf, gloss_hbm.at[b, pl.ds(base, CAP)])

            def clear(r, carry):
                for k in range(NK):
                    staging[r, pl.ds(k * 16, 16)] = jnp.zeros((16,), jnp.float32)
                cnt[r] = 0
                return carry
            lax.fori_loop(0, nused, clear, 0)
            for k in range(CAP // 16):
                gbuf[pl.ds(k * 16, 16)] = jnp.zeros((16,), jnp.int32)
            state[2] = base + CAP

        @pl.loop(0, T // FCH)
        def _(ci):
            t0 = pl.multiple_of(ci * FCH, FCH)
            pltpu.sync_copy(rep_hbm.at[pl.ds(t0, FCH), b], repv)

            @pl.loop(0, FCH)
            def _(j):
                p = _lane0(predv[pl.ds(t0 + j, 16)])
                kept = p != BLANK
                is_start = jnp.logical_and(kept, p != state[1])

                @pl.when(is_start)
                def _():
                    @pl.when(state[0] == CAP - 1)
                    def _():
                        flush(CAP)
                        state[0] = -1
                    r = state[0] + 1
                    state[0] = r
                    state[1] = p
                    plsc.store_scatter(gbuf, [lane_iota * 0 + r],
                                       lane_iota * 0 + p, mask=lane0_mask)

                @pl.when(kept)
                def _():
                    r = state[0]
                    cnt[r] = cnt[r] + 1
                    for k in range(NK):
                        sl = pl.ds(k * 16, 16)
                        staging[r, sl] = staging[r, sl] + repv[j, sl]

        nused = state[0] + 1
        length = state[2] + nused
        lenv[pl.ds(0, 16)] = jnp.zeros((16,), jnp.int32) + length
        pltpu.sync_copy(lenv, len_hbm.at[b])
        flush(nused)
        ntail = (T - state[2]) // CAP

        def tail(i, carry):
            flush(0)
            return carry
        lax.fori_loop(0, ntail, tail, 0)


@jax.jit
def _shrink(representation, logit, pad32):
    pred = pl.pallas_call(
        _argmax_body,
        grid=(T // TBLK,),
        in_specs=[
            pl.BlockSpec((TBLK, B, C), lambda i: (i, 0, 0)),
            pl.BlockSpec((B, TBLK), lambda i: (0, i)),
        ],
        out_specs=pl.BlockSpec((B, TBLK), lambda i: (0, i)),
        out_shape=jax.ShapeDtypeStruct((B, T), jnp.int32),
    )(logit, pad32)

    mesh = plsc.VectorSubcoreMesh(core_axis_name="c", subcore_axis_name="s")
    sc_kernel = pl.kernel(
        _sc_shrink_body,
        out_type=(
            jax.ShapeDtypeStruct((T, B, D), jnp.float32),
            jax.ShapeDtypeStruct((B, T), jnp.int32),
            jax.ShapeDtypeStruct((B, 16), jnp.int32),
        ),
        mesh=mesh,
        compiler_params=pltpu.CompilerParams(use_tc_tiling_on_sc=False,
                                             needs_layout_passes=False),
        scratch_types=[
            pltpu.VMEM((FCH, D), jnp.float32),
            pltpu.VMEM((T + 16,), jnp.int32),
            pltpu.VMEM((CAP, D), jnp.float32),
            pltpu.VMEM((CAP,), jnp.int32),
            pltpu.SMEM((CAP,), jnp.int32),
            pltpu.VMEM((16,), jnp.int32),
            pltpu.SMEM((4,), jnp.int32),
        ],
    )
    out, gloss, lenpad = sc_kernel(representation, pred)

    padmask = pl.pallas_call(
        _padmask_body,
        in_specs=[pl.BlockSpec((B, 16), lambda: (0, 0))],
        out_specs=pl.BlockSpec((B, T), lambda: (0, 0)),
        out_shape=jax.ShapeDtypeStruct((B, T), jnp.int32),
    )(lenpad)
    return out, padmask, gloss, lenpad


def kernel(representation, logit, padding):
    pad32 = padding.astype(jnp.int32)
    out, padmask, gloss, lenpad = _shrink(representation, logit, pad32)
    return (out, padmask.astype(jnp.bool_), gloss, lenpad[:, 0])


# speedup vs baseline: 3.4368x; 3.4368x over previous
import jax
import jax.numpy as jnp
from jax import lax
from jax.experimental import pallas as pl
from jax.experimental.pallas import tpu as pltpu
from jax.experimental.pallas import tpu_sc as plsc

T, B, D, C = 2048, 8, 512, 1000
BLANK = 0
TBLK = 128
NK = D // 16
RCH = 64
QROWS = T // 4


def _argmax_body(logit_ref, pad_ref, pred_ref):
    x = logit_ref[...]
    m = jnp.max(x, axis=-1, keepdims=True)
    iota = lax.broadcasted_iota(jnp.int32, x.shape, 2)
    idx = jnp.min(jnp.where(x >= m, iota, C), axis=-1)
    pad = pad_ref[...]
    idx_t = jnp.transpose(idx, (1, 0))
    pred_ref[...] = jnp.where(pad != 0, BLANK, idx_t).astype(jnp.int32)


def _padmask_body(len_ref, mask_ref):
    l = len_ref[...][:, 0:1]
    iota = lax.broadcasted_iota(jnp.int32, (B, T), 1)
    mask_ref[...] = (iota >= l).astype(jnp.int32)


def _lane_shift_right(v, lanes):
    idx = jnp.maximum(lanes - 1, 0)
    return lax.gather(
        v, idx[:, None],
        dimension_numbers=lax.GatherDimensionNumbers(
            offset_dims=(), collapsed_slice_dims=(0,), start_index_map=(0,)),
        slice_sizes=(1,),
        mode=lax.GatherScatterMode.PROMISE_IN_BOUNDS)


def _sc_scan_body(pred_hbm, srcidx_hbm, cnts_hbm, kept_hbm, q_hbm,
                  gloss_hbm, small_hbm,
                  predv, srcv, glossv, qv, keptv, cntv, smallv, st):
    c = lax.axis_index("c")
    s = lax.axis_index("s")
    w = c * 16 + s
    lanes = lax.iota(jnp.int32, 16)
    zi = jnp.zeros((16,), jnp.int32)

    @pl.when(w < B)
    def _():
        b = w
        pltpu.sync_copy(pred_hbm.at[b], predv.at[pl.ds(0, T)])

        @pl.loop(0, T // 16)
        def _(k):
            sl = pl.ds(pl.multiple_of(k * 16, 16), 16)
            srcv[sl] = zi
            glossv[sl] = zi
        st[0] = 0
        st[1] = 0
        st[2] = -1

        @pl.loop(0, T // 16)
        def _(j):
            t0 = pl.multiple_of(j * 16, 16)
            pv = predv[pl.ds(t0, 16)]
            kept = pv != BLANK
            tvec = t0 + lanes
            code = jnp.where(kept, tvec * 4096 + pv, -1)
            carry = zi + st[2]
            call = jnp.maximum(plsc.cummax(code), carry)
            prev = jnp.where(lanes == 0, carry, _lane_shift_right(call, lanes))
            prevlab = prev & 4095
            start = jnp.logical_and(
                kept, jnp.logical_or(pv != prevlab, prev < 0))
            ki = kept.astype(jnp.int32)
            nbex = plsc.cumsum(ki) - ki + st[1]
            ns = st[0]
            nk = st[1]
            plsc.store_compressed(srcv.at[pl.ds(ns, 16)],
                                  tvec * B + b, mask=start)
            plsc.store_compressed(glossv.at[pl.ds(ns, 16)], pv, mask=start)
            plsc.store_compressed(qv.at[pl.ds(ns, 16)], nbex, mask=start)
            plsc.store_compressed(keptv.at[pl.ds(nk, 16)],
                                  tvec * B + b, mask=kept)
            st[0] = ns + plsc.all_reduce_population_count(start)[0]
            st[1] = nk + plsc.all_reduce_population_count(kept)[0]
            st[2] = call[15]

        nruns = st[0]
        nkept = st[1]
        plsc.store_scatter(qv, [zi + nruns], zi + nkept, mask=lanes == 0)

        @pl.loop(0, T // 16)
        def _(k):
            s0 = pl.multiple_of(k * 16, 16)
            q0 = qv[pl.ds(s0, 16)]
            q1 = qv[pl.ds(s0 + 1, 16)]
            cb = jnp.where(s0 + lanes < nruns, q1 - q0, 1)
            cntv[pl.ds(s0, 16)] = cb

        smallv[pl.ds(0, 16)] = jnp.where(lanes == 0, zi + nruns,
                                         jnp.where(lanes == 1, zi + nkept, zi))
        pltpu.sync_copy(srcv.at[pl.ds(0, T)], srcidx_hbm.at[b])
        pltpu.sync_copy(cntv.at[pl.ds(0, T)], cnts_hbm.at[b])
        pltpu.sync_copy(keptv.at[pl.ds(0, T)], kept_hbm.at[b])
        pltpu.sync_copy(qv, q_hbm.at[b])
        pltpu.sync_copy(glossv.at[pl.ds(0, T)], gloss_hbm.at[b])
        pltpu.sync_copy(smallv, small_hbm.at[b])


def _sc_move_body(rep_hbm, srcidx_hbm, cnts_hbm, kept_hbm, q_hbm, small_hbm,
                  out_hbm,
                  idxv, rowbuf, zerobuf, accv, rowtmp, cntv, qv, fixv,
                  keptv, smallv, st, sem):
    c = lax.axis_index("c")
    s = lax.axis_index("s")
    w = c * 16 + s
    b = w // 4
    quarter = w % 4
    lo = pl.multiple_of(quarter * QROWS, RCH)
    lanes = lax.iota(jnp.int32, 16)
    zf = jnp.zeros((16,), jnp.float32)
    zi = jnp.zeros((16,), jnp.int32)

    pltpu.sync_copy(small_hbm.at[b], smallv)
    nruns = smallv[pl.ds(0, 16)][0]

    @pl.loop(0, RCH)
    def _(r):
        for k in range(NK):
            zerobuf[r, pl.ds(k * 16, 16)] = zf

    @pl.loop(0, QROWS // RCH)
    def _(ch):
        r0 = pl.multiple_of(lo + ch * RCH, RCH)
        nh = jnp.maximum(jnp.minimum(nruns - r0, RCH), 0)

        @pl.when(nh > 0)
        def _():
            pltpu.sync_copy(srcidx_hbm.at[b, pl.ds(r0, RCH)], idxv)
            pltpu.sync_copy(rep_hbm.at[idxv], rowbuf)

            def zrow(r, carry):
                for k in range(NK):
                    rowbuf[r, pl.ds(k * 16, 16)] = zf
                return carry
            lax.fori_loop(nh, RCH, zrow, 0)
            pltpu.sync_copy(rowbuf, out_hbm.at[pl.ds(r0, RCH), b])

        @pl.when(nh <= 0)
        def _():
            pltpu.sync_copy(zerobuf, out_hbm.at[pl.ds(r0, RCH), b])

    pltpu.sync_copy(cnts_hbm.at[b, pl.ds(lo, QROWS)], cntv.at[pl.ds(0, QROWS)])
    pltpu.sync_copy(q_hbm.at[b, pl.ds(lo, QROWS + 16)],
                    qv.at[pl.ds(0, QROWS + 16)])
    st[0] = 0

    @pl.loop(0, QROWS // 16)
    def _(k):
        s0 = pl.multiple_of(k * 16, 16)
        cb = cntv[pl.ds(s0, 16)]
        m = cb > 1
        off = st[0]
        plsc.store_compressed(fixv.at[pl.ds(off, 16)],
                              lo + s0 + lanes, mask=m)
        st[0] = off + plsc.all_reduce_population_count(m)[0]
    nfix = st[0]

    @pl.when(nfix > 0)
    def _():
        pltpu.sync_copy(kept_hbm.at[b], keptv.at[pl.ds(0, T)])

        def fix(i, carry):
            sg = fixv[pl.ds(i, 16)][0]
            sl = sg - lo
            cnt = cntv[pl.ds(sl, 16)][0]
            q0 = qv[pl.ds(sl, 16)][0]
            for k in range(NK):
                accv[pl.ds(k * 16, 16)] = zf

            def inner(f, c2):
                kidx = keptv[pl.ds(q0 + f, 16)][0]
                pltpu.sync_copy(rep_hbm.at[kidx], rowtmp)
                for k in range(NK):
                    ksl = pl.ds(k * 16, 16)
                    accv[ksl] = accv[ksl] + rowtmp[ksl]
                return c2
            lax.fori_loop(0, cnt, inner, 0)
            rcp = 1.0 / (zf + cnt.astype(jnp.float32))
            for k in range(NK):
                ksl = pl.ds(k * 16, 16)
                accv[ksl] = accv[ksl] * rcp
            pltpu.sync_copy(accv, out_hbm.at[sg, b])
            return carry
        lax.fori_loop(0, nfix, fix, 0)


@jax.jit
def _shrink(representation, logit, pad32):
    pred = pl.pallas_call(
        _argmax_body,
        grid=(T // TBLK,),
        in_specs=[
            pl.BlockSpec((TBLK, B, C), lambda i: (i, 0, 0)),
            pl.BlockSpec((B, TBLK), lambda i: (0, i)),
        ],
        out_specs=pl.BlockSpec((B, TBLK), lambda i: (0, i)),
        out_shape=jax.ShapeDtypeStruct((B, T), jnp.int32),
    )(logit, pad32)

    mesh = plsc.VectorSubcoreMesh(core_axis_name="c", subcore_axis_name="s")
    cparams = pltpu.CompilerParams(use_tc_tiling_on_sc=False,
                                   needs_layout_passes=False)

    scan_kernel = pl.kernel(
        _sc_scan_body,
        out_type=(
            jax.ShapeDtypeStruct((B, T), jnp.int32),
            jax.ShapeDtypeStruct((B, T), jnp.int32),
            jax.ShapeDtypeStruct((B, T), jnp.int32),
            jax.ShapeDtypeStruct((B, T + 32), jnp.int32),
            jax.ShapeDtypeStruct((B, T), jnp.int32),
            jax.ShapeDtypeStruct((B, 16), jnp.int32),
        ),
        mesh=mesh,
        compiler_params=cparams,
        scratch_types=[
            pltpu.VMEM((T + 16,), jnp.int32),
            pltpu.VMEM((T + 16,), jnp.int32),
            pltpu.VMEM((T + 16,), jnp.int32),
            pltpu.VMEM((T + 32,), jnp.int32),
            pltpu.VMEM((T + 16,), jnp.int32),
            pltpu.VMEM((T + 16,), jnp.int32),
            pltpu.VMEM((16,), jnp.int32),
            pltpu.SMEM((4,), jnp.int32),
        ],
    )
    srcidx, cnts, kept, q, gloss, small = scan_kernel(pred)

    rep1d = representation.reshape(T * B, D)
    move_kernel = pl.kernel(
        _sc_move_body,
        out_type=jax.ShapeDtypeStruct((T, B, D), jnp.float32),
        mesh=mesh,
        compiler_params=cparams,
        scratch_types=[
            pltpu.VMEM((RCH,), jnp.int32),
            pltpu.VMEM((RCH, D), jnp.float32),
            pltpu.VMEM((RCH, D), jnp.float32),
            pltpu.VMEM((D,), jnp.float32),
            pltpu.VMEM((D,), jnp.float32),
            pltpu.VMEM((QROWS + 32,), jnp.int32),
            pltpu.VMEM((QROWS + 32,), jnp.int32),
            pltpu.VMEM((QROWS + 16,), jnp.int32),
            pltpu.VMEM((T + 16,), jnp.int32),
            pltpu.VMEM((16,), jnp.int32),
            pltpu.SMEM((4,), jnp.int32),
            pltpu.SemaphoreType.DMA,
        ],
    )
    out = move_kernel(rep1d, srcidx, cnts, kept, q, small)

    padmask = pl.pallas_call(
        _padmask_body,
        in_specs=[pl.BlockSpec((B, 16), lambda: (0, 0))],
        out_specs=pl.BlockSpec((B, T), lambda: (0, 0)),
        out_shape=jax.ShapeDtypeStruct((B, T), jnp.int32),
    )(small)
    return out, padmask, gloss, small


def kernel(representation, logit, padding):
    pad32 = padding.astype(jnp.int32)
    out, padmask, gloss, small = _shrink(representation, logit, pad32)
    return (out, padmask.astype(jnp.bool_), gloss, small[:, 0])
